# Initial kernel scaffold; baseline (speedup 1.0000x reference)
import jax
import jax.numpy as jnp
from jax.experimental import pallas as pl


def kernel(x, theta_w, theta_b, phi_w, phi_b, g_w, g_b, W_w, W_b, bn_gamma, bn_beta, bn_mean, bn_var):
    raise NotImplementedError("write your pallas kernel here")


# speedup vs baseline: 1.7592x; 1.7592x over previous
import jax
import jax.numpy as jnp
from jax import lax
from jax.experimental import pallas as pl
from jax.experimental.pallas import tpu as pltpu


def _pool2x2(ref, H, W):
    a = jnp.maximum(ref[0::2, :], ref[1::2, :])
    a4 = a.reshape(H // 2, 2, W // 2, ref.shape[-1])
    c = jnp.max(a4, axis=1)
    return c.reshape(-1, ref.shape[-1])


def _fused_kernel(x_ref, wpg_ref, bpg_ref, wfold_ref, wtheta_ref,
                  btheta_ref, bfoldT_ref, o_ref, phi_ref, g_ref, *, H, W, D):
    x = x_ref[...]
    pg = lax.dot_general(
        x, wpg_ref[...], (((0,), (0,)), ((), ())),
        preferred_element_type=jnp.float32)
    phi_ref[...] = pg[:, :D]
    g_ref[...] = pg[:, D:]
    bpg = bpg_ref[...]
    phi = _pool2x2(phi_ref, H, W) + bpg[:, :D]
    g = _pool2x2(g_ref, H, W) + bpg[:, D:]
    m = lax.dot_general(
        phi, g, (((0,), (0,)), ((), ())),
        preferred_element_type=jnp.float32)
    wb = jnp.dot(m, wfold_ref[...],
                 preferred_element_type=jnp.float32)
    wcT = lax.dot_general(
        wb, wtheta_ref[...], (((0,), (1,)), ((), ())),
        preferred_element_type=jnp.float32)
    bcT = lax.dot_general(
        wb, btheta_ref[...], (((0,), (1,)), ((), ())),
        preferred_element_type=jnp.float32)
    o_ref[...] = (
        jnp.dot(wcT, x, preferred_element_type=jnp.float32)
        + bcT + bfoldT_ref[...] + x
    ).astype(o_ref.dtype)


@jax.jit
def kernel(x, theta_w, theta_b, phi_w, phi_b, g_w, g_b, W_w, W_b,
           bn_gamma, bn_beta, bn_mean, bn_var):
    B, C, H, W = x.shape
    D = theta_w.shape[1]
    N = H * W
    Nkv = (H // 2) * (W // 2)

    x3 = x.reshape(B, C, N)
    w_pg = jnp.concatenate([phi_w, g_w], axis=1)
    b_pg = jnp.concatenate([phi_b, g_b])[None, :]

    eps = 1e-5
    scale = bn_gamma / jnp.sqrt(bn_var + eps)
    w_fold_s = (W_w * scale[None, :]) * (1.0 / Nkv)
    b_fold = (W_b - bn_mean) * scale + bn_beta

    import functools
    z = pl.pallas_call(
        functools.partial(_fused_kernel, H=H, W=W, D=D),
        out_shape=jax.ShapeDtypeStruct((B, C, N), x.dtype),
        grid=(B,),
        in_specs=[
            pl.BlockSpec((None, C, N), lambda b: (b, 0, 0)),
            pl.BlockSpec((C, 2 * D), lambda b: (0, 0)),
            pl.BlockSpec((1, 2 * D), lambda b: (0, 0)),
            pl.BlockSpec((D, C), lambda b: (0, 0)),
            pl.BlockSpec((C, D), lambda b: (0, 0)),
            pl.BlockSpec((1, D), lambda b: (0, 0)),
            pl.BlockSpec((C, 1), lambda b: (0, 0)),
        ],
        out_specs=pl.BlockSpec((None, C, N), lambda b: (b, 0, 0)),
        scratch_shapes=[pltpu.VMEM((N, D), jnp.float32),
                        pltpu.VMEM((N, D), jnp.float32)],
        compiler_params=pltpu.CompilerParams(
            dimension_semantics=("parallel",)),
    )(x3, w_pg, b_pg, w_fold_s, theta_w, theta_b[None, :], b_fold[:, None])
    return z.reshape(B, C, H, W)
